# baseline (speedup 1.0000x reference)
import jax
import jax.numpy as jnp
from jax.experimental import pallas as pl


def _gru_block(h_ref, v_ref, wz_ref, wr_ref, wq_ref, o_ref):
    h = h_ref[...]
    v = v_ref[...]
    Wz = wz_ref[...]
    Wr = wr_ref[...]
    Wq = wq_ref[...]
    D = h.shape[1]

    def mm(a, b):
        return jax.lax.dot_general(
            a, b, (((1,), (0,)), ((), ())), preferred_element_type=jnp.float32
        )

    z = jax.nn.sigmoid(mm(h, Wz[:D]) + mm(v, Wz[D:]))
    r = jax.nn.sigmoid(mm(h, Wr[:D]) + mm(v, Wr[D:]))
    q = jnp.tanh(mm(r * h, Wq[:D]) + mm(v, Wq[D:]))
    o_ref[...] = (1.0 - z) * h + z * q


def kernel(mem, val, Wz, Wr, Wq, idx):
    M, D = mem.shape
    B = idx.shape[0]
    idx = idx.astype(jnp.int32)
    h = jnp.take(mem, idx, axis=0)

    BLK = 6912
    grid = (B // BLK,)
    h_new = pl.pallas_call(
        _gru_block,
        grid=grid,
        in_specs=[
            pl.BlockSpec((BLK, D), lambda i: (i, 0)),
            pl.BlockSpec((BLK, D), lambda i: (i, 0)),
            pl.BlockSpec((2 * D, D), lambda i: (0, 0)),
            pl.BlockSpec((2 * D, D), lambda i: (0, 0)),
            pl.BlockSpec((2 * D, D), lambda i: (0, 0)),
        ],
        out_specs=pl.BlockSpec((BLK, D), lambda i: (i, 0)),
        out_shape=jax.ShapeDtypeStruct((B, D), jnp.float32),
    )(h, val, Wz, Wr, Wq)

    pos = jnp.arange(B, dtype=jnp.int32)
    win = jnp.zeros((M,), jnp.int32).at[idx].max(pos)
    keep = win[idx] == pos
    idx_eff = jnp.where(keep, idx, M)
    return mem.at[idx_eff].set(h_new, mode="drop")


# speedup vs baseline: 2.2243x; 2.2243x over previous
import functools

import jax
import jax.numpy as jnp
from jax import lax
from jax.experimental import pallas as pl
from jax.experimental.pallas import tpu as pltpu
from jax.experimental.pallas import tpu_sc as plsc

_NC = 2
_NS = 16
_NW = _NC * _NS


def _wid():
    return lax.axis_index("s") * _NC + lax.axis_index("c")


def _gru_block(h_ref, v_ref, wz_ref, wr_ref, wq_ref, o_ref):
    h = h_ref[...]
    v = v_ref[...]
    Wz = wz_ref[...]
    Wr = wr_ref[...]
    Wq = wq_ref[...]
    D = h.shape[1]

    def mm(a, b):
        return jax.lax.dot_general(
            a, b, (((1,), (0,)), ((), ())), preferred_element_type=jnp.float32
        )

    z = jax.nn.sigmoid(mm(h, Wz[:D]) + mm(v, Wz[D:]))
    r = jax.nn.sigmoid(mm(h, Wr[:D]) + mm(v, Wr[D:]))
    q = jnp.tanh(mm(r * h, Wq[:D]) + mm(v, Wq[D:]))
    o_ref[...] = (1.0 - z) * h + z * q


@functools.cache
def _make_gather(M, B, D, W):
    CPW = B // _NW
    NWIN = CPW // W
    K = W // 128
    mesh = plsc.VectorSubcoreMesh(core_axis_name="c", subcore_axis_name="s")

    @functools.partial(
        pl.kernel,
        out_type=jax.ShapeDtypeStruct((B, D), jnp.float32),
        mesh=mesh,
        scratch_types=[
            pltpu.VMEM((K, 128), jnp.int32),
            pltpu.VMEM((W, D), jnp.float32),
            pltpu.SemaphoreType.DMA,
        ],
        compiler_params=pltpu.CompilerParams(use_tc_tiling_on_sc=False),
    )
    def gather(mem_hbm, idx_hbm, out_hbm, idx_v, rows_v, sem):
        w = _wid()
        rbase = w * (CPW // 128)

        @pl.loop(0, NWIN)
        def _win(g):
            r0 = rbase + g * K
            pltpu.sync_copy(idx_hbm.at[pl.ds(r0, K)], idx_v)
            cps = [
                pltpu.async_copy(
                    mem_hbm.at[idx_v.at[j]],
                    rows_v.at[pl.ds(j * 128, 128)],
                    sem,
                )
                for j in range(K)
            ]
            for cp in cps:
                cp.wait()
            pltpu.sync_copy(rows_v, out_hbm.at[pl.ds(r0 * 128, W)])

    return gather


@functools.cache
def _make_scatter(M, B, D, KF, W):
    CPW = B // _NW
    NWIN = CPW // W
    K = W // 128
    mesh = plsc.VectorSubcoreMesh(core_axis_name="c", subcore_axis_name="s")

    @functools.partial(
        pl.kernel,
        out_type=(),
        mesh=mesh,
        scratch_types=[
            pltpu.VMEM((K, 128), jnp.int32),
            pltpu.VMEM((W, D), jnp.float32),
            pltpu.VMEM((8, KF), jnp.int32),
            pltpu.VMEM((KF, D), jnp.float32),
            pltpu.SemaphoreType.DMA,
        ],
        compiler_params=pltpu.CompilerParams(use_tc_tiling_on_sc=False),
    )
    def scatter(mem_ref, idx_hbm, upd_hbm, fidx_hbm, fval_hbm,
                idx_v, rows_v, fidx_v, fval_v, sem):
        w = _wid()
        rbase = w * (CPW // 128)

        @pl.loop(0, NWIN)
        def _win(g):
            r0 = rbase + g * K
            pltpu.sync_copy(idx_hbm.at[pl.ds(r0, K)], idx_v)
            pltpu.sync_copy(upd_hbm.at[pl.ds(r0 * 128, W)], rows_v)
            cps = [
                pltpu.async_copy(
                    rows_v.at[pl.ds(j * 128, 128)],
                    mem_ref.at[idx_v.at[j]],
                    sem,
                )
                for j in range(K)
            ]
            for cp in cps:
                cp.wait()

        pltpu.sync_copy(fidx_hbm.at[w], fidx_v)
        pltpu.sync_copy(fval_hbm.at[w], fval_v)
        pltpu.async_copy(fval_v, mem_ref.at[fidx_v.at[0]], sem).wait()

    return scatter


def kernel(mem, val, Wz, Wr, Wq, idx):
    M, D = mem.shape
    B = idx.shape[0]
    idx = idx.astype(jnp.int32)
    W = 512
    KF = 64
    CPW = B // _NW

    h = _make_gather(M, B, D, W)(mem, idx.reshape(B // 128, 128))

    BLK = 6912
    h_new = pl.pallas_call(
        _gru_block,
        grid=(B // BLK,),
        in_specs=[
            pl.BlockSpec((BLK, D), lambda i: (i, 0)),
            pl.BlockSpec((BLK, D), lambda i: (i, 0)),
            pl.BlockSpec((2 * D, D), lambda i: (0, 0)),
            pl.BlockSpec((2 * D, D), lambda i: (0, 0)),
            pl.BlockSpec((2 * D, D), lambda i: (0, 0)),
        ],
        out_specs=pl.BlockSpec((BLK, D), lambda i: (i, 0)),
        out_shape=jax.ShapeDtypeStruct((B, D), jnp.float32),
    )(h, val, Wz, Wr, Wq)

    pos = jnp.arange(B, dtype=jnp.int32)
    win = jnp.zeros((M,), jnp.int32).at[idx].max(pos)
    keep = win[idx] == pos
    tail_pos = (pos // CPW + 1) * CPW - KF + (pos % KF)
    idx_eff = jnp.where(keep, idx, idx[tail_pos])
    fpos = (jnp.arange(_NW, dtype=jnp.int32)[:, None] + 1) * CPW - KF \
        + jnp.arange(KF, dtype=jnp.int32)[None, :]
    fidx = idx[fpos]
    fval = h_new[win[fidx]]
    fidx3 = jnp.broadcast_to(fidx[:, None, :], (_NW, 8, KF))

    mem_ref = jax.new_ref(mem)
    _make_scatter(M, B, D, KF, W)(
        mem_ref, idx_eff.reshape(B // 128, 128), h_new, fidx3, fval)
    return mem_ref[...]
